# Initial kernel scaffold; baseline (speedup 1.0000x reference)
import jax
import jax.numpy as jnp
from jax.experimental import pallas as pl


def kernel(x, table):
    raise NotImplementedError("write your pallas kernel here")


# speedup vs baseline: 1.8716x; 1.8716x over previous
import functools

import jax
import jax.numpy as jnp
from jax import lax
from jax.experimental import pallas as pl
from jax.experimental.pallas import tpu as pltpu
from jax.experimental.pallas import tpu_sc as plsc

D = 64
GRP = 128
NC, NS = 2, 16
NW = NC * NS
GPB = 5
CH = GPB * GRP
NBUF = 2


def _body(x_hbm, table_hbm, out_hbm, idx_v, rows0, rows1, sg0, sg1, so0, so1):
    wid = lax.axis_index("s") * NC + lax.axis_index("c")
    gw = x_hbm.shape[0] // NW
    n_chunks = gw // GPB
    row_base = wid * gw * GRP

    pltpu.sync_copy(x_hbm.at[pl.ds(wid * gw, gw)], idx_v)

    rows = (rows0, rows1)
    sg = (sg0, sg1)
    so = (so0, so1)

    def issue_gathers(g, b):
        for j in range(GPB):
            pltpu.async_copy(
                table_hbm.at[idx_v.at[g * GPB + j]],
                rows[b].at[pl.ds(j * GRP, GRP)],
                sg[b])

    def drain_gathers(b):
        for j in range(GPB):
            pltpu.make_async_copy(
                table_hbm.at[idx_v.at[0]],
                rows[b].at[pl.ds(j * GRP, GRP)],
                sg[b]).wait()

    def writeout(g, b):
        dst = out_hbm.at[pl.ds(row_base + g * CH, CH)]
        pltpu.async_copy(rows[b], dst, so[b])
        pltpu.make_async_copy(rows[b], dst, so[b]).wait()

    for b in range(NBUF):
        issue_gathers(b, b)

    @pl.loop(0, n_chunks // NBUF - 1)
    def _steady(g0):
        for b in range(NBUF):
            g = g0 * NBUF + b
            drain_gathers(b)
            writeout(g, b)
            issue_gathers(g + NBUF, b)

    for b in range(NBUF):
        drain_gathers(b)
        writeout(n_chunks - NBUF + b, b)


def kernel(x, table):
    B, H = x.shape
    BT = B * H
    xf = x.astype(jnp.int32).reshape(BT // GRP, GRP)
    grid_kernel = pl.kernel(
        _body,
        out_type=jax.ShapeDtypeStruct((BT, D), jnp.float32),
        mesh=plsc.VectorSubcoreMesh(core_axis_name="c", subcore_axis_name="s"),
        compiler_params=pltpu.CompilerParams(use_tc_tiling_on_sc=False),
        scratch_types=[
            pltpu.VMEM((BT // GRP // NW, GRP), jnp.int32),
            pltpu.VMEM((CH, D), jnp.float32),
            pltpu.VMEM((CH, D), jnp.float32),
            pltpu.SemaphoreType.DMA,
            pltpu.SemaphoreType.DMA,
            pltpu.SemaphoreType.DMA,
            pltpu.SemaphoreType.DMA,
        ],
    )
    out = grid_kernel(xf, table)
    return out.reshape(B, H, D)


# speedup vs baseline: 1.8847x; 1.0070x over previous
import jax
import jax.numpy as jnp
from jax import lax
from jax.experimental import pallas as pl
from jax.experimental.pallas import tpu as pltpu
from jax.experimental.pallas import tpu_sc as plsc

D = 64
GRP = 128
NC, NS = 2, 16
NW = NC * NS
GPB = 4
CH = GPB * GRP
NBUF = 3


def _body(x_hbm, table_hbm, out_hbm, idx_v,
          rows0, rows1, rows2, sg0, sg1, sg2, so0, so1, so2):
    wid = lax.axis_index("s") * NC + lax.axis_index("c")
    gw = x_hbm.shape[0] // NW
    n_chunks = gw // GPB
    row_base = wid * gw * GRP

    pltpu.sync_copy(x_hbm.at[pl.ds(wid * gw, gw)], idx_v)

    rows = (rows0, rows1, rows2)
    sg = (sg0, sg1, sg2)
    so = (so0, so1, so2)

    def issue_gathers(g, b):
        for j in range(GPB):
            pltpu.async_copy(
                table_hbm.at[idx_v.at[g * GPB + j]],
                rows[b].at[pl.ds(j * GRP, GRP)],
                sg[b])

    def drain_gathers(b):
        for j in range(GPB):
            pltpu.make_async_copy(
                table_hbm.at[idx_v.at[0]],
                rows[b].at[pl.ds(j * GRP, GRP)],
                sg[b]).wait()

    def issue_writeout(g, b):
        pltpu.async_copy(rows[b], out_hbm.at[pl.ds(row_base + g * CH, CH)],
                         so[b])

    def drain_writeout(b):
        pltpu.make_async_copy(rows[b], out_hbm.at[pl.ds(row_base, CH)],
                              so[b]).wait()

    def step(g, b, wait_prev=True, issue_next=True):
        drain_gathers(b)
        issue_writeout(g, b)
        if issue_next:
            bn = (b + 2) % NBUF
            if wait_prev:
                drain_writeout(bn)
            issue_gathers(g + 2, bn)

    issue_gathers(0, 0)
    issue_gathers(1, 1)
    step(0, 0, wait_prev=False)

    steady = (n_chunks - 3) // NBUF

    @pl.loop(0, steady)
    def _steady(t):
        for k in range(NBUF):
            g = 1 + t * NBUF + k
            step(g, (1 + k) % NBUF)

    for g in range(1 + steady * NBUF, n_chunks - 2):
        step(g, g % NBUF)
    for g in range(n_chunks - 2, n_chunks):
        step(g, g % NBUF, issue_next=False)

    for b in range(NBUF):
        drain_writeout(b)


def kernel(x, table):
    B, H = x.shape
    BT = B * H
    xf = x.astype(jnp.int32).reshape(BT // GRP, GRP)
    grid_kernel = pl.kernel(
        _body,
        out_type=jax.ShapeDtypeStruct((BT, D), jnp.float32),
        mesh=plsc.VectorSubcoreMesh(core_axis_name="c", subcore_axis_name="s"),
        compiler_params=pltpu.CompilerParams(use_tc_tiling_on_sc=False),
        scratch_types=[
            pltpu.VMEM((BT // GRP // NW, GRP), jnp.int32),
            pltpu.VMEM((CH, D), jnp.float32),
            pltpu.VMEM((CH, D), jnp.float32),
            pltpu.VMEM((CH, D), jnp.float32),
            pltpu.SemaphoreType.DMA,
            pltpu.SemaphoreType.DMA,
            pltpu.SemaphoreType.DMA,
            pltpu.SemaphoreType.DMA,
            pltpu.SemaphoreType.DMA,
            pltpu.SemaphoreType.DMA,
        ],
    )
    out = grid_kernel(xf, table)
    return out.reshape(B, H, D)
